# Initial kernel scaffold; baseline (speedup 1.0000x reference)
import jax
import jax.numpy as jnp
from jax.experimental import pallas as pl


def kernel(x, edge_index, W, b, eps):
    raise NotImplementedError("write your pallas kernel here")


# speedup vs baseline: 7.4311x; 7.4311x over previous
import functools

import jax
import jax.numpy as jnp
from jax import lax
from jax.experimental import pallas as pl
from jax.experimental.pallas import tpu as pltpu
from jax.experimental.pallas import tpu_sc as plsc

N = 10000
E = 320000
D = 128

NUM_CORES = 2
NUM_SUBCORES = 16
NUM_TILES = NUM_CORES * NUM_SUBCORES
EDGES_PER_TILE = E // NUM_TILES
CHUNK = 80
CHUNKS_PER_TILE = EDGES_PER_TILE // CHUNK
ROWS_PER_TILE = 640
N_PAD = ROWS_PER_TILE * NUM_SUBCORES


def _sc_aggregate():
    mesh = plsc.VectorSubcoreMesh(core_axis_name="c", subcore_axis_name="s")

    @functools.partial(
        pl.kernel,
        mesh=mesh,
        out_type=jax.ShapeDtypeStruct((NUM_CORES, NUM_SUBCORES, ROWS_PER_TILE, D),
                                      jnp.float32),
        scratch_types=[
            pltpu.VMEM((CHUNKS_PER_TILE, CHUNK), jnp.int32),
            pltpu.VMEM((CHUNKS_PER_TILE, CHUNK), jnp.int32),
            pltpu.VMEM((CHUNK, D), jnp.float32),
            pltpu.VMEM_SHARED((N_PAD, D), jnp.float32),
            pltpu.SemaphoreType.DMA,
        ],
    )
    def sc_agg(src_hbm, dst_hbm, x_hbm, zeros_hbm, out_hbm,
               src_v, dst_v, rows_v, acc_sh, sem):
        cid = lax.axis_index("c")
        sid = lax.axis_index("s")
        wid = cid * NUM_SUBCORES + sid
        row_off = pl.multiple_of(sid * ROWS_PER_TILE, 8)

        pltpu.sync_copy(zeros_hbm, rows_v)
        for j in range(ROWS_PER_TILE // CHUNK):
            pltpu.sync_copy(rows_v, acc_sh.at[pl.ds(row_off + j * CHUNK, CHUNK)])
        plsc.subcore_barrier()

        pltpu.sync_copy(src_hbm.at[wid], src_v)
        pltpu.sync_copy(dst_hbm.at[wid], dst_v)

        def body(j, carry):
            pltpu.async_copy(x_hbm.at[src_v.at[j]], rows_v, sem).wait()
            pltpu.sync_copy(rows_v, acc_sh.at[dst_v.at[j]], add=True)
            return carry

        lax.fori_loop(0, CHUNKS_PER_TILE, body, 0)
        plsc.subcore_barrier()

        for j in range(ROWS_PER_TILE // CHUNK):
            pltpu.sync_copy(acc_sh.at[pl.ds(row_off + j * CHUNK, CHUNK)], rows_v)
            pltpu.sync_copy(rows_v, out_hbm.at[cid, sid, pl.ds(j * CHUNK, CHUNK)])

    return sc_agg


_SC_AGG = _sc_aggregate()


def _tc_fn(x_ref, a0_ref, a1_ref, w_ref, b_ref, s_ref, o_ref):
    h = s_ref[0, 0] * x_ref[...] + a0_ref[...] + a1_ref[...]
    o = jnp.dot(h, w_ref[...], preferred_element_type=jnp.float32) + b_ref[...]
    o_ref[...] = jnp.maximum(o, 0.0)


ROW_BLK = 1000


def _tc_dense(x, a0, a1, W, b2, scale):
    return pl.pallas_call(
        _tc_fn,
        grid=(N // ROW_BLK,),
        in_specs=[
            pl.BlockSpec((ROW_BLK, D), lambda i: (i, 0)),
            pl.BlockSpec((ROW_BLK, D), lambda i: (i, 0)),
            pl.BlockSpec((ROW_BLK, D), lambda i: (i, 0)),
            pl.BlockSpec((D, D), lambda i: (0, 0)),
            pl.BlockSpec((1, D), lambda i: (0, 0)),
            pl.BlockSpec(memory_space=pltpu.SMEM),
        ],
        out_specs=pl.BlockSpec((ROW_BLK, D), lambda i: (i, 0)),
        out_shape=jax.ShapeDtypeStruct((N, D), jnp.float32),
    )(x, a0, a1, W, b2, scale)


def kernel(x, edge_index, W, b, eps):
    src = edge_index[0].reshape(NUM_TILES, CHUNKS_PER_TILE, CHUNK)
    dst = edge_index[1].reshape(NUM_TILES, CHUNKS_PER_TILE, CHUNK)
    zeros = jnp.zeros((CHUNK, D), jnp.float32)
    partials = _SC_AGG(src, dst, x, zeros)
    partials = partials.reshape(NUM_CORES, N_PAD, D)[:, :N, :]
    scale = (1.0 + eps).astype(jnp.float32).reshape(1, 1)
    return _tc_dense(x, partials[0], partials[1], W, b.reshape(1, D), scale)
